# baseline (speedup 1.0000x reference)
import functools

import jax
import jax.numpy as jnp
from jax import lax
from jax.experimental import pallas as pl
from jax.experimental.pallas import tpu as pltpu
from jax.experimental.pallas import tpu_sc as plsc

LATENT = 32
NUM_WORKERS = 32
LANES = 16
NCOLS = 7813
COLS_PER_W = 256
SCOLS = 8
N_STRIPES = COLS_PER_W // SCOLS
MAX_START = 7812 * 128


def _make_extract(batch, n_val):
    mesh = plsc.VectorSubcoreMesh(core_axis_name="c", subcore_axis_name="s")
    cparams = pltpu.CompilerParams(needs_layout_passes=False)
    qcap = batch + LANES
    n_groups_all = batch // LANES

    @functools.partial(
        pl.kernel,
        out_type=(
            jax.ShapeDtypeStruct((batch + NUM_WORKERS, 2, 128), jnp.float32),
            jax.ShapeDtypeStruct((batch + NUM_WORKERS, 2, 128), jnp.float32),
        ),
        mesh=mesh,
        compiler_params=cparams,
        scratch_types=[
            pltpu.VMEM((batch,), jnp.int32),
            pltpu.VMEM((qcap,), jnp.int32),
            pltpu.VMEM((qcap,), jnp.int32),
            pltpu.VMEM((SCOLS, LATENT, 128), jnp.float32),
            pltpu.VMEM((LANES, 2, 128), jnp.float32),
            pltpu.SemaphoreType.DMA,
            pltpu.SemaphoreType.DMA,
        ],
    )
    def extract(uidx_hbm, iidx_hbm, u_tab, i_tab, stage_u, stage_i,
                idx_v, bq_v, pq_v, sbuf_v, rb_v, sem, ssem):
        wid = lax.axis_index("s") * 2 + lax.axis_index("c")
        iota = lax.broadcasted_iota(jnp.int32, (LANES,), 0)
        zi = jnp.zeros((LANES,), jnp.int32)
        trash = batch + wid

        for idx_hbm, tab, stage in ((uidx_hbm, u_tab, stage_u),
                                    (iidx_hbm, i_tab, stage_i)):
            pltpu.sync_copy(idx_hbm, idx_v)

            def scan_body(v, off):
                bvec = idx_v[pl.ds(v * LANES, LANES)]
                mine = (bvec >> 15) == wid
                pos = zi + v * LANES + iota
                plsc.store_compressed(bq_v.at[pl.ds(off, LANES)], bvec,
                                      mask=mine)
                plsc.store_compressed(pq_v.at[pl.ds(off, LANES)], pos,
                                      mask=mine)
                return off + plsc.all_reduce_population_count(mine)[0]

            nq = lax.fori_loop(0, n_groups_all, scan_body, jnp.int32(0))
            ngroups = (nq + LANES - 1) // LANES

            @pl.loop(0, N_STRIPES)
            def _(st):
                base_col = wid * COLS_PER_W + st * SCOLS
                copies = []
                for t in range(SCOLS):
                    start = pl.multiple_of(
                        jnp.minimum((base_col + t) * 128, MAX_START), 128)
                    copies.append(pltpu.async_copy(
                        tab.at[:, pl.ds(start, 128)], sbuf_v.at[t], sem))
                for cp in copies:
                    cp.wait()

                def group_body(g, carry, base_col=base_col):
                    bvec = bq_v[pl.ds(g * LANES, LANES)]
                    pvec = pq_v[pl.ds(g * LANES, LANES)]
                    j = bvec >> 7
                    mask = ((j >= base_col) & (j < base_col + SCOLS)
                            & (g * LANES + iota < nq))
                    cnt = plsc.all_reduce_population_count(mask)[0]

                    @pl.when(cnt > 0)
                    def _():
                        slot = j - base_col
                        lane = bvec - jnp.minimum(j * 128, MAX_START)
                        for d in range(LATENT):
                            vals = plsc.load_gather(
                                sbuf_v, [slot, zi + d, lane], mask=mask)
                            plsc.store_scatter(
                                rb_v, [iota, zi, zi + d], vals, mask=mask)
                        pos_c = jnp.where(mask, pvec, trash)
                        pltpu.async_copy(rb_v, stage.at[pos_c], ssem).wait()

                    return carry

                lax.fori_loop(0, ngroups, group_body, jnp.int32(0))

    return extract


def _make_pair(batch):
    mesh = plsc.VectorSubcoreMesh(core_axis_name="c", subcore_axis_name="s")
    cparams = pltpu.CompilerParams(needs_layout_passes=False)
    b_per_w = batch // NUM_WORKERS
    chunk = 128
    n_chunks = b_per_w // chunk

    @functools.partial(
        pl.kernel,
        out_type=jax.ShapeDtypeStruct((batch,), jnp.float32),
        mesh=mesh,
        compiler_params=cparams,
        scratch_types=[
            pltpu.VMEM((chunk, 2, 128), jnp.float32),
            pltpu.VMEM((chunk, 2, 128), jnp.float32),
            pltpu.VMEM((b_per_w,), jnp.float32),
            pltpu.SemaphoreType.DMA,
        ],
    )
    def pair(stage_u, stage_i, out_hbm, ub_v, ib_v, out_v, sem):
        wid = lax.axis_index("s") * 2 + lax.axis_index("c")
        base = wid * b_per_w
        iota = lax.broadcasted_iota(jnp.int32, (LANES,), 0)
        zi = jnp.zeros((LANES,), jnp.int32)

        for c in range(n_chunks):
            cp_u = pltpu.async_copy(
                stage_u.at[pl.ds(base + c * chunk, chunk)], ub_v, sem)
            cp_i = pltpu.async_copy(
                stage_i.at[pl.ds(base + c * chunk, chunk)], ib_v, sem)
            cp_u.wait()
            cp_i.wait()
            for g in range(chunk // LANES):
                rows = iota + g * LANES
                acc = jnp.zeros((LANES,), jnp.float32)
                for d in range(LATENT):
                    u_d = plsc.load_gather(ub_v, [rows, zi, zi + d])
                    i_d = plsc.load_gather(ib_v, [rows, zi, zi + d])
                    acc = acc + u_d * i_d
                out_v[pl.ds(c * chunk + g * LANES, LANES)] = acc

        pltpu.sync_copy(out_v, out_hbm.at[pl.ds(base, b_per_w)])

    return pair


def kernel(user, item, U, I):
    batch = user.shape[0]
    uidx = user.reshape(-1).astype(jnp.int32)
    iidx = item.reshape(-1).astype(jnp.int32)
    stage_u, stage_i = _make_extract(batch, U.shape[0])(uidx, iidx, U.T, I.T)
    out = _make_pair(batch)(stage_u, stage_i)
    return out.reshape(batch, 1)


# speedup vs baseline: 3.9686x; 3.9686x over previous
import functools

import jax
import jax.numpy as jnp
from jax import lax
from jax.experimental import pallas as pl
from jax.experimental.pallas import tpu as pltpu
from jax.experimental.pallas import tpu_sc as plsc

LATENT = 32
NUM_WORKERS = 32
LANES = 16
WAVE = 8


def _make_kernel(batch: int):
    b_per_w = batch // NUM_WORKERS
    n_super = b_per_w // LANES
    mesh = plsc.VectorSubcoreMesh(core_axis_name="c", subcore_axis_name="s")
    cparams = pltpu.CompilerParams(needs_layout_passes=False)

    @functools.partial(
        pl.kernel,
        out_type=jax.ShapeDtypeStruct((batch,), jnp.float32),
        mesh=mesh,
        compiler_params=cparams,
        scratch_types=[
            pltpu.VMEM((b_per_w,), jnp.int32),
            pltpu.VMEM((b_per_w,), jnp.int32),
            pltpu.VMEM((WAVE, LATENT, 128), jnp.float32),
            pltpu.VMEM((WAVE, LATENT, 128), jnp.float32),
            pltpu.VMEM((LANES, LANES), jnp.float32),
            pltpu.VMEM((b_per_w,), jnp.float32),
            pltpu.SemaphoreType.DMA,
            pltpu.SemaphoreType.DMA,
        ],
    )
    def bpr_kernel(user_hbm, item_hbm, ut_hbm, it_hbm, out_hbm,
                   uidx_v, iidx_v, ubuf_v, ibuf_v, tbuf_v, out_v,
                   idx_sem, sem):
        wid = lax.axis_index("s") * 2 + lax.axis_index("c")
        cp_u = pltpu.async_copy(user_hbm.at[wid], uidx_v, idx_sem)
        cp_i = pltpu.async_copy(item_hbm.at[wid], iidx_v, idx_sem)
        cp_u.wait()
        cp_i.wait()

        iota = lax.broadcasted_iota(jnp.int32, (LANES,), 0)
        zeros_i = jnp.zeros((LANES,), jnp.int32)

        @pl.loop(0, n_super)
        def _(sb):
            base = sb * LANES
            uvec = uidx_v[pl.ds(base, LANES)]
            ivec = iidx_v[pl.ds(base, LANES)]
            for half in range(2):
                copies = []
                for kk in range(WAVE):
                    k = half * WAVE + kk
                    bu = uvec[k]
                    bi = ivec[k]
                    su = pl.multiple_of((bu // 128) * 128, 128)
                    si = pl.multiple_of((bi // 128) * 128, 128)
                    copies.append(pltpu.async_copy(
                        ut_hbm.at[:, pl.ds(su, 128)], ubuf_v.at[kk], sem))
                    copies.append(pltpu.async_copy(
                        it_hbm.at[:, pl.ds(si, 128)], ibuf_v.at[kk], sem))
                for cp in copies:
                    cp.wait()
                for kk in range(WAVE):
                    k = half * WAVE + kk
                    lu = zeros_i + (uvec[k] % 128)
                    li = zeros_i + (ivec[k] % 128)
                    u_lo = plsc.load_gather(ubuf_v.at[kk], [iota, lu])
                    u_hi = plsc.load_gather(ubuf_v.at[kk], [iota + 16, lu])
                    i_lo = plsc.load_gather(ibuf_v.at[kk], [iota, li])
                    i_hi = plsc.load_gather(ibuf_v.at[kk], [iota + 16, li])
                    p = u_lo * i_lo + u_hi * i_hi
                    plsc.store_scatter(tbuf_v, [iota, zeros_i + k], p)
            acc = jnp.zeros((LANES,), jnp.float32)
            for r in range(LANES):
                acc = acc + tbuf_v.at[r][...]
            out_v[pl.ds(base, LANES)] = acc

        pltpu.sync_copy(out_v, out_hbm.at[pl.ds(wid * b_per_w, b_per_w)])

    return bpr_kernel


def kernel(user, item, U, I):
    batch = user.shape[0]
    b_per_w = batch // NUM_WORKERS
    uidx = user.reshape(-1).astype(jnp.int32).reshape(NUM_WORKERS, b_per_w)
    iidx = item.reshape(-1).astype(jnp.int32).reshape(NUM_WORKERS, b_per_w)
    out = _make_kernel(batch)(uidx, iidx, U.T, I.T)
    return out.reshape(batch, 1)


# speedup vs baseline: 4.6647x; 1.1754x over previous
import functools

import jax
import jax.numpy as jnp
from jax import lax
from jax.experimental import pallas as pl
from jax.experimental.pallas import tpu as pltpu
from jax.experimental.pallas import tpu_sc as plsc

LATENT = 32
NUM_WORKERS = 32
LANES = 16
WAVE = 4


def _make_kernel(batch: int):
    b_per_w = batch // NUM_WORKERS
    n_super = b_per_w // LANES
    mesh = plsc.VectorSubcoreMesh(core_axis_name="c", subcore_axis_name="s")
    cparams = pltpu.CompilerParams(needs_layout_passes=False)

    @functools.partial(
        pl.kernel,
        out_type=jax.ShapeDtypeStruct((batch,), jnp.float32),
        mesh=mesh,
        compiler_params=cparams,
        scratch_types=[
            pltpu.VMEM((b_per_w,), jnp.int32),
            pltpu.VMEM((b_per_w,), jnp.int32),
            pltpu.VMEM((2, WAVE, LATENT, 128), jnp.float32),
            pltpu.VMEM((2, WAVE, LATENT, 128), jnp.float32),
            pltpu.VMEM((LANES, LANES), jnp.float32),
            pltpu.VMEM((b_per_w,), jnp.float32),
            pltpu.SemaphoreType.DMA,
            pltpu.SemaphoreType.DMA,
        ],
    )
    def bpr_kernel(user_hbm, item_hbm, ut_hbm, it_hbm, out_hbm,
                   uidx_v, iidx_v, ubuf_v, ibuf_v, tbuf_v, out_v,
                   idx_sem, sem):
        wid = lax.axis_index("s") * 2 + lax.axis_index("c")
        cp_u = pltpu.async_copy(user_hbm.at[wid], uidx_v, idx_sem)
        cp_i = pltpu.async_copy(item_hbm.at[wid], iidx_v, idx_sem)
        cp_u.wait()
        cp_i.wait()

        iota = lax.broadcasted_iota(jnp.int32, (LANES,), 0)
        zeros_i = jnp.zeros((LANES,), jnp.int32)
        n_waves = LANES // WAVE

        @pl.loop(0, n_super)
        def _(sb):
            base = sb * LANES
            uvec = uidx_v[pl.ds(base, LANES)]
            ivec = iidx_v[pl.ds(base, LANES)]

            def fire(w):
                buf = w % 2
                copies = []
                for kk in range(WAVE):
                    k = w * WAVE + kk
                    su = pl.multiple_of((uvec[k] // 128) * 128, 128)
                    si = pl.multiple_of((ivec[k] // 128) * 128, 128)
                    copies.append(pltpu.async_copy(
                        ut_hbm.at[:, pl.ds(su, 128)],
                        ubuf_v.at[buf, kk], sem))
                    copies.append(pltpu.async_copy(
                        it_hbm.at[:, pl.ds(si, 128)],
                        ibuf_v.at[buf, kk], sem))
                return copies

            def extract(w):
                buf = w % 2
                for kk in range(WAVE):
                    k = w * WAVE + kk
                    lu = zeros_i + (uvec[k] % 128)
                    li = zeros_i + (ivec[k] % 128)
                    u_lo = plsc.load_gather(ubuf_v.at[buf, kk], [iota, lu])
                    u_hi = plsc.load_gather(ubuf_v.at[buf, kk],
                                            [iota + 16, lu])
                    i_lo = plsc.load_gather(ibuf_v.at[buf, kk], [iota, li])
                    i_hi = plsc.load_gather(ibuf_v.at[buf, kk],
                                            [iota + 16, li])
                    p = u_lo * i_lo + u_hi * i_hi
                    plsc.store_scatter(tbuf_v, [iota, zeros_i + k], p)

            pending = fire(0)
            for w in range(n_waves):
                nxt = fire(w + 1) if w + 1 < n_waves else []
                for cp in pending:
                    cp.wait()
                extract(w)
                pending = nxt

            acc = jnp.zeros((LANES,), jnp.float32)
            for r in range(LANES):
                acc = acc + tbuf_v.at[r][...]
            out_v[pl.ds(base, LANES)] = acc

        pltpu.sync_copy(out_v, out_hbm.at[pl.ds(wid * b_per_w, b_per_w)])

    return bpr_kernel


def kernel(user, item, U, I):
    batch = user.shape[0]
    b_per_w = batch // NUM_WORKERS
    uidx = user.reshape(-1).astype(jnp.int32).reshape(NUM_WORKERS, b_per_w)
    iidx = item.reshape(-1).astype(jnp.int32).reshape(NUM_WORKERS, b_per_w)
    out = _make_kernel(batch)(uidx, iidx, U.T, I.T)
    return out.reshape(batch, 1)
